# Initial kernel scaffold; baseline (speedup 1.0000x reference)
import jax
import jax.numpy as jnp
from jax.experimental import pallas as pl


def kernel(x, edge_index, edge_attr, W1, b1, W2, b2, W3, b3, W4, b4, W5, b5, g1, be1, g2, be2, g3, be3, g4, be4):
    raise NotImplementedError("write your pallas kernel here")


# speedup vs baseline: 7.9992x; 7.9992x over previous
import functools

import jax
import jax.numpy as jnp
from jax import lax
from jax.experimental import pallas as pl
from jax.experimental.pallas import tpu as pltpu
from jax.experimental.pallas import tpu_sc as plsc

NN = 10000
NP = 10240
EE = 320000
NSUB = 16
CHUNK = 128
NCHUNK = 79
EPAD = 2 * NSUB * NCHUNK * CHUNK
BN_EPS = 1e-3

_mesh = plsc.VectorSubcoreMesh(core_axis_name="c", subcore_axis_name="s")

_BCAST_DNUMS = lax.GatherDimensionNumbers(
    offset_dims=(), collapsed_slice_dims=(0,), start_index_map=(0,))


def _bcast_lane(v16, lane):
    idx = jnp.full((16, 1), lane, jnp.int32)
    return lax.gather(v16, idx, _BCAST_DNUMS, (1,),
                      mode=lax.GatherScatterMode.PROMISE_IN_BOUNDS)


@functools.partial(
    pl.kernel,
    out_type=jax.ShapeDtypeStruct((2, NSUB, 640), jnp.float32),
    mesh=_mesh,
    scratch_types=[
        pltpu.VMEM((NCHUNK, CHUNK), jnp.int32),
        pltpu.VMEM((NCHUNK, CHUNK), jnp.float32),
        pltpu.VMEM((640,), jnp.float32),
        pltpu.VMEM_SHARED((NP,), jnp.float32),
    ],
)
def _deg_kernel(col_hbm, ew_hbm, out_hbm, col_v, ew_v, zb, acc):
    cid = lax.axis_index("c")
    sid = lax.axis_index("s")
    wid = cid * NSUB + sid
    zeros16 = jnp.zeros((16,), jnp.float32)

    def zb_body(i, _):
        zb[pl.ds(i * 16, 16)] = zeros16
        return 0

    lax.fori_loop(0, 640 // 16, zb_body, 0)
    pltpu.sync_copy(zb, acc.at[pl.ds(sid * 640, 640)])
    pltpu.sync_copy(col_hbm.at[wid], col_v)
    pltpu.sync_copy(ew_hbm.at[wid], ew_v)
    plsc.subcore_barrier()

    def chunk_body(j, _):
        pltpu.sync_copy(ew_v.at[j], acc.at[col_v.at[j]], add=True)
        return 0

    lax.fori_loop(0, NCHUNK, chunk_body, 0)
    plsc.subcore_barrier()
    pltpu.sync_copy(acc.at[pl.ds(sid * 640, 640)], out_hbm.at[cid, sid])


@functools.partial(
    pl.kernel,
    out_type=jax.ShapeDtypeStruct((2, NSUB, 640, CHUNK), jnp.float32),
    mesh=_mesh,
    scratch_types=[
        pltpu.VMEM((NCHUNK, CHUNK), jnp.int32),
        pltpu.VMEM((NCHUNK, CHUNK), jnp.int32),
        pltpu.VMEM((NCHUNK, CHUNK), jnp.float32),
        pltpu.VMEM((CHUNK, CHUNK), jnp.float32),
        pltpu.VMEM_SHARED((NP, CHUNK), jnp.float32),
        pltpu.SemaphoreType.DMA,
    ],
)
def _agg_kernel(us_hbm, row_hbm, col_hbm, ew_hbm, out_hbm,
                row_v, col_v, ew_v, rows_v, acc, sem):
    cid = lax.axis_index("c")
    sid = lax.axis_index("s")
    wid = cid * NSUB + sid
    zeros16 = jnp.zeros((16,), jnp.float32)

    def zrow(i, _):
        for f in range(CHUNK // 16):
            rows_v[i, pl.ds(f * 16, 16)] = zeros16
        return 0

    lax.fori_loop(0, CHUNK, zrow, 0)
    for r in range(640 // CHUNK):
        pltpu.sync_copy(rows_v, acc.at[pl.ds(sid * 640 + r * CHUNK, CHUNK)])
    pltpu.sync_copy(row_hbm.at[wid], row_v)
    pltpu.sync_copy(col_hbm.at[wid], col_v)
    pltpu.sync_copy(ew_hbm.at[wid], ew_v)
    plsc.subcore_barrier()

    def chunk_body(j, _):
        pltpu.async_copy(us_hbm.at[row_v.at[j]], rows_v, sem).wait()

        def escale(g, _):
            ew16 = ew_v[j, pl.ds(g * 16, 16)]
            for e16 in range(16):
                w = _bcast_lane(ew16, e16)
                e = g * 16 + e16
                for f in range(CHUNK // 16):
                    sl = pl.ds(f * 16, 16)
                    rows_v[e, sl] = rows_v[e, sl] * w
            return 0

        lax.fori_loop(0, CHUNK // 16, escale, 0)
        pltpu.sync_copy(rows_v, acc.at[col_v.at[j]], add=True)
        return 0

    lax.fori_loop(0, NCHUNK, chunk_body, 0)
    plsc.subcore_barrier()
    pltpu.sync_copy(acc.at[pl.ds(sid * 640, 640)], out_hbm.at[cid, sid])


def _aggregate(us, row_t, col_t, ew_t):
    o = _agg_kernel(us, row_t, col_t, ew_t)
    o = o.reshape(2, NP, CHUNK)[:, :NN]
    return o[0] + o[1]


def _agg_full(u, dinv, row_t, col_t, ew_t):
    us = u * dinv[:, None]
    blocks = []
    for f in range(u.shape[1] // CHUNK):
        usf = us[:, f * CHUNK:(f + 1) * CHUNK]
        blocks.append(_aggregate(usf, row_t, col_t, ew_t) + usf)
    acc = jnp.concatenate(blocks, axis=1) if len(blocks) > 1 else blocks[0]
    return acc * dinv[:, None]


def _batch_norm(z, g, be):
    mu = z.mean(axis=0)
    var = z.var(axis=0)
    return (z - mu) * lax.rsqrt(var + BN_EPS) * g + be


def kernel(x, edge_index, edge_attr, W1, b1, W2, b2, W3, b3, W4, b4, W5, b5,
           g1, be1, g2, be2, g3, be3, g4, be4):
    pad = EPAD - EE
    row = jnp.concatenate([edge_index[0], jnp.zeros((pad,), jnp.int32)])
    col = jnp.concatenate([edge_index[1], jnp.zeros((pad,), jnp.int32)])
    ew = jnp.concatenate([edge_attr, jnp.zeros((pad,), jnp.float32)])
    row_t = row.reshape(2 * NSUB, NCHUNK, CHUNK)
    col_t = col.reshape(2 * NSUB, NCHUNK, CHUNK)
    ew_t = ew.reshape(2 * NSUB, NCHUNK, CHUNK)

    degp = _deg_kernel(col_t, ew_t)
    deg = degp.reshape(2, NP)[:, :NN].sum(axis=0) + 1.0
    dinv = lax.rsqrt(deg)

    agg = lambda u: _agg_full(u, dinv, row_t, col_t, ew_t)

    h = jax.nn.sigmoid(_batch_norm(agg(x) @ W1 + b1, g1, be1))
    h = jax.nn.sigmoid(_batch_norm(agg(h) @ W2 + b2, g2, be2))
    h = jax.nn.sigmoid(_batch_norm(agg(h @ W3) + b3, g3, be3))
    h = jax.nn.sigmoid(_batch_norm(agg(h @ W4) + b4, g4, be4))
    h = jax.nn.sigmoid(agg(h @ W5) + b5)

    A = h.T @ h
    A = (A + A.T) / 2.0
    A = A - jnp.diag(jnp.diag(A))
    return A
